# baseline (speedup 1.0000x reference)
import functools

import jax
import jax.numpy as jnp
from jax import lax
from jax.experimental import pallas as pl
from jax.experimental.pallas import tpu as pltpu
from jax.experimental.pallas import tpu_sc as plsc


def _make_gather(V, D, B, NC, NS):
    NW = NC * NS
    b_per_w = B // NW
    CHUNK = 128
    K = 8
    STEP = CHUNK * K
    n_chunks = b_per_w // CHUNK
    n_steps = b_per_w // STEP
    assert B % NW == 0 and b_per_w % STEP == 0

    mesh = plsc.VectorSubcoreMesh(core_axis_name="c", subcore_axis_name="s")

    @functools.partial(
        pl.kernel,
        out_type=jax.ShapeDtypeStruct((B, D), jnp.float32),
        mesh=mesh,
        scratch_types=[
            pltpu.VMEM((n_chunks, CHUNK), jnp.int32),
            pltpu.VMEM((STEP, D), jnp.float32),
            pltpu.SemaphoreType.DMA,
        ],
        compiler_params=pltpu.CompilerParams(use_tc_tiling_on_sc=False),
    )
    def gather_kernel(idx_hbm, table_hbm, out_hbm, idx_v, rows_v, sem):
        wid = lax.axis_index("s") * NC + lax.axis_index("c")
        base = wid * b_per_w
        pltpu.sync_copy(idx_hbm.at[wid], idx_v)

        def step(g, _):
            cps = []
            for b in range(K):
                cps.append(
                    pltpu.async_copy(
                        table_hbm.at[idx_v.at[g * K + b]],
                        rows_v.at[pl.ds(b * CHUNK, CHUNK)],
                        sem,
                    )
                )
            for cp in cps:
                cp.wait()
            pltpu.sync_copy(rows_v, out_hbm.at[pl.ds(base + g * STEP, STEP)])
            return ()

        lax.fori_loop(0, n_steps, step, (), unroll=False)

    return gather_kernel


def kernel(x, table):
    B0, S = x.shape
    V, D = table.shape
    B = B0 * S
    info = plsc.get_sparse_core_info()
    NC, NS = info.num_cores, info.num_subcores
    NW = NC * NS
    idx = x.reshape(NW, (B // NW) // 128, 128).astype(jnp.int32)
    out = _make_gather(V, D, B, NC, NS)(idx, table)
    return out.reshape(B0, S, D)


# speedup vs baseline: 1.0090x; 1.0090x over previous
import functools

import jax
import jax.numpy as jnp
from jax import lax
from jax.experimental import pallas as pl
from jax.experimental.pallas import tpu as pltpu
from jax.experimental.pallas import tpu_sc as plsc


def _make_gather(V, D, B, NC, NS):
    NW = NC * NS
    b_per_w = B // NW
    CHUNK = 128
    K = 5
    STEP = CHUNK * K
    n_chunks = b_per_w // CHUNK
    n_steps = b_per_w // STEP
    n_outer = n_steps // 2
    assert B % NW == 0 and b_per_w % STEP == 0 and n_steps % 2 == 0

    mesh = plsc.VectorSubcoreMesh(core_axis_name="c", subcore_axis_name="s")

    @functools.partial(
        pl.kernel,
        out_type=jax.ShapeDtypeStruct((B, D), jnp.float32),
        mesh=mesh,
        scratch_types=[
            pltpu.VMEM((n_chunks, CHUNK), jnp.int32),
            pltpu.VMEM((STEP, D), jnp.float32),
            pltpu.VMEM((STEP, D), jnp.float32),
            pltpu.SemaphoreType.DMA,
            pltpu.SemaphoreType.DMA,
            pltpu.SemaphoreType.DMA,
        ],
        compiler_params=pltpu.CompilerParams(use_tc_tiling_on_sc=False),
    )
    def gather_kernel(idx_hbm, table_hbm, out_hbm, idx_v, rows0, rows1,
                      sem_g, sem_o0, sem_o1):
        wid = lax.axis_index("s") * NC + lax.axis_index("c")
        base = wid * b_per_w
        pltpu.sync_copy(idx_hbm.at[wid], idx_v)

        bufs = ((0, rows0, sem_o0), (1, rows1, sem_o1))

        def pair(t, _):
            for b, buf, sem_o in bufs:
                g = t * 2 + b

                @pl.when(t > 0)
                def _wait_prev_drain():
                    pltpu.make_async_copy(
                        buf, out_hbm.at[pl.ds(base, STEP)], sem_o
                    ).wait()

                cps = []
                for i in range(K):
                    cps.append(
                        pltpu.async_copy(
                            table_hbm.at[idx_v.at[g * K + i]],
                            buf.at[pl.ds(i * CHUNK, CHUNK)],
                            sem_g,
                        )
                    )
                for cp in cps:
                    cp.wait()
                pltpu.async_copy(
                    buf, out_hbm.at[pl.ds(base + g * STEP, STEP)], sem_o
                )
            return ()

        lax.fori_loop(0, n_outer, pair, (), unroll=False)
        for _, buf, sem_o in bufs:
            pltpu.make_async_copy(
                buf, out_hbm.at[pl.ds(base, STEP)], sem_o
            ).wait()

    return gather_kernel


def kernel(x, table):
    B0, S = x.shape
    V, D = table.shape
    B = B0 * S
    info = plsc.get_sparse_core_info()
    NC, NS = info.num_cores, info.num_subcores
    NW = NC * NS
    idx = x.reshape(NW, (B // NW) // 128, 128).astype(jnp.int32)
    out = _make_gather(V, D, B, NC, NS)(idx, table)
    return out.reshape(B0, S, D)
